# baseline (speedup 1.0000x reference)
import jax
import jax.numpy as jnp
from jax.experimental import pallas as pl


def _chamfer_body(p_ref, g_ref, p2g_ref, g2p_ref):
    p = p_ref[0]
    g = g_ref[0]
    psq = jnp.sum(p * p, axis=0, keepdims=True)
    gsq = jnp.sum(g * g, axis=0, keepdims=True)
    a = jnp.concatenate([-2.0 * p, psq, jnp.ones_like(psq)], axis=0)
    b = jnp.concatenate([g, jnp.ones_like(gsq), gsq], axis=0)
    d = jax.lax.dot_general(
        a, b, (((0,), (0,)), ((), ())),
        precision=jax.lax.Precision.HIGHEST,
        preferred_element_type=jnp.float32,
    )
    p2g_ref[0] = jnp.sqrt(jnp.mean(jnp.min(d, axis=1))).reshape(1, 1)
    g2p_ref[0] = jnp.sqrt(jnp.mean(jnp.min(d, axis=0))).reshape(1, 1)


def kernel(points, gts):
    bs, n, _ = points.shape
    m = gts.shape[1]
    pts_t = jnp.transpose(points, (0, 2, 1))
    gts_t = jnp.transpose(gts, (0, 2, 1))
    p2g_b, g2p_b = pl.pallas_call(
        _chamfer_body,
        grid=(bs,),
        in_specs=[
            pl.BlockSpec((1, 3, n), lambda b: (b, 0, 0)),
            pl.BlockSpec((1, 3, m), lambda b: (b, 0, 0)),
        ],
        out_specs=[
            pl.BlockSpec((1, 1, 1), lambda b: (b, 0, 0)),
            pl.BlockSpec((1, 1, 1), lambda b: (b, 0, 0)),
        ],
        out_shape=[
            jax.ShapeDtypeStruct((bs, 1, 1), jnp.float32),
            jax.ShapeDtypeStruct((bs, 1, 1), jnp.float32),
        ],
    )(pts_t, gts_t)
    p2g = jnp.mean(p2g_b)
    g2p = jnp.mean(g2p_b)
    loss = (p2g + g2p) / 2.0
    return (loss, p2g, g2p)


# speedup vs baseline: 1.8266x; 1.8266x over previous
import jax
import jax.numpy as jnp
from jax.experimental import pallas as pl

_CHUNK = 512


def _chamfer_body(p_ref, g_ref, p2g_ref, g2p_ref):
    pts = p_ref[0]
    g = g_ref[0]
    n = pts.shape[0]
    m = g.shape[1]
    px = pts[:, 0:1]
    py = pts[:, 1:2]
    pz = pts[:, 2:3]
    rowmin = None
    g2p_sum = None
    for k in range(0, m, _CHUNK):
        gx = g[0:1, k:k + _CHUNK]
        gy = g[1:2, k:k + _CHUNK]
        gz = g[2:3, k:k + _CHUNK]
        dx = px - gx
        dy = py - gy
        dz = pz - gz
        d = dx * dx + dy * dy + dz * dz
        rm = jnp.min(d, axis=1, keepdims=True)
        rowmin = rm if rowmin is None else jnp.minimum(rowmin, rm)
        cs = jnp.sum(jnp.min(d, axis=0))
        g2p_sum = cs if g2p_sum is None else g2p_sum + cs
    p2g_ref[0] = jnp.sqrt(jnp.mean(rowmin)).reshape(1, 1)
    g2p_ref[0] = jnp.sqrt(g2p_sum / m).reshape(1, 1)


def kernel(points, gts):
    bs, n, _ = points.shape
    m = gts.shape[1]
    gts_t = jnp.transpose(gts, (0, 2, 1))
    p2g_b, g2p_b = pl.pallas_call(
        _chamfer_body,
        grid=(bs,),
        in_specs=[
            pl.BlockSpec((1, n, 3), lambda b: (b, 0, 0)),
            pl.BlockSpec((1, 3, m), lambda b: (b, 0, 0)),
        ],
        out_specs=[
            pl.BlockSpec((1, 1, 1), lambda b: (b, 0, 0)),
            pl.BlockSpec((1, 1, 1), lambda b: (b, 0, 0)),
        ],
        out_shape=[
            jax.ShapeDtypeStruct((bs, 1, 1), jnp.float32),
            jax.ShapeDtypeStruct((bs, 1, 1), jnp.float32),
        ],
    )(points, gts_t)
    p2g = jnp.mean(p2g_b)
    g2p = jnp.mean(g2p_b)
    loss = (p2g + g2p) / 2.0
    return (loss, p2g, g2p)
